# baseline (speedup 1.0000x reference)
import functools

import jax
import jax.numpy as jnp
from jax import lax
from jax.experimental import pallas as pl
from jax.experimental.pallas import tpu as pltpu
from jax.experimental.pallas import tpu_sc as plsc

_H = 128
_SEGMENTS = 4
_TILE = 3200


@functools.lru_cache(maxsize=None)
def _make_gather(num_tokens, seg, s):
    info = plsc.get_sparse_core_info()
    nc, ns = info.num_cores, info.num_subcores
    nw = nc * ns
    assert seg % nw == 0
    b_per_w = seg // nw
    chunk = 400
    assert b_per_w % chunk == 0
    n_chunks = b_per_w // chunk
    mesh = plsc.VectorSubcoreMesh(core_axis_name="c", subcore_axis_name="s")

    @functools.partial(
        pl.kernel,
        mesh=mesh,
        out_type=jax.ShapeDtypeStruct((seg, _H), jnp.float32),
        scratch_types=[
            pltpu.VMEM((chunk,), jnp.int32),
            pltpu.VMEM((chunk,), jnp.int32),
            pltpu.VMEM((chunk, _H), jnp.float32),
            pltpu.VMEM((chunk, _H), jnp.float32),
            pltpu.SemaphoreType.DMA,
            pltpu.SemaphoreType.DMA,
            pltpu.SemaphoreType.DMA,
            pltpu.SemaphoreType.DMA,
            pltpu.SemaphoreType.DMA,
            pltpu.SemaphoreType.DMA,
        ],
    )
    def gather(table_hbm, ids_hbm, out_hbm, idx0, idx1, rows0, rows1,
               isem0, isem1, gsem0, gsem1, wsem0, wsem1):
        wid = lax.axis_index("s") * nc + lax.axis_index("c")
        base = wid * b_per_w
        idxs = (idx0, idx1)
        bufs = (rows0, rows1)
        isems = (isem0, isem1)
        gsems = (gsem0, gsem1)
        wsems = (wsem0, wsem1)
        ih = [None] * n_chunks
        gh = [None] * n_chunks
        wh = [None] * n_chunks

        def start_i(i):
            ih[i] = pltpu.async_copy(
                ids_hbm.at[pl.ds(s * seg + base + i * chunk, chunk)],
                idxs[i % 2], isems[i % 2])

        def start_g(i):
            gh[i] = pltpu.async_copy(
                table_hbm.at[idxs[i % 2]], bufs[i % 2], gsems[i % 2])

        def start_w(i):
            wh[i] = pltpu.async_copy(
                bufs[i % 2], out_hbm.at[pl.ds(base + i * chunk, chunk)],
                wsems[i % 2])

        start_i(0)
        ih[0].wait()
        start_g(0)
        if n_chunks > 1:
            start_i(1)
        for i in range(n_chunks):
            gh[i].wait()
            if i >= 1:
                wh[i - 1].wait()
            if i + 2 < n_chunks:
                start_i(i + 2)
            if i + 1 < n_chunks:
                ih[i + 1].wait()
                start_g(i + 1)
            start_w(i)
        wh[n_chunks - 1].wait()

    return gather


def _mm_body(seqt_ref, g_ref, wf_ref, bf_ref, wc_ref, bc_ref, *rest):
    out_ref, weff_ref, beff_ref = rest[-3], rest[-2], rest[-1]
    wtop = wc_ref[:_H, :]

    @pl.when(pl.program_id(0) == 0)
    def _fold_weights():
        weff_ref[...] = lax.dot(
            wf_ref[...], wtop, precision=lax.Precision.HIGHEST,
            preferred_element_type=jnp.float32)
        beff_ref[...] = lax.dot(
            bf_ref[...], wtop, precision=lax.Precision.HIGHEST,
            preferred_element_type=jnp.float32) + bc_ref[...]

    wbot = wc_ref[_H:, :]
    out_ref[...] = (
        lax.dot_general(seqt_ref[...], weff_ref[:7, :],
                        (((0,), (0,)), ((), ())),
                        precision=lax.Precision.HIGHEST,
                        preferred_element_type=jnp.float32)
        + lax.dot(g_ref[...].astype(jnp.bfloat16),
                  wbot.astype(jnp.bfloat16),
                  preferred_element_type=jnp.float32)
        + beff_ref[...])


def _fused_matmul_seg(seqt, g, wf8, bf2, W_cat, bc2, out_prev, s):
    t = seqt.shape[1]
    seg = g.shape[0]
    assert seg % _TILE == 0
    n_tiles = seg // _TILE
    off = s * n_tiles
    in_specs = [
        pl.BlockSpec((7, _TILE), lambda i: (0, off + i)),
        pl.BlockSpec((_TILE, _H), lambda i: (i, 0)),
        pl.BlockSpec((8, _H), lambda i: (0, 0)),
        pl.BlockSpec((1, _H), lambda i: (0, 0)),
        pl.BlockSpec((2 * _H, _H), lambda i: (0, 0)),
        pl.BlockSpec((1, _H), lambda i: (0, 0)),
    ]
    inputs = [seqt, g, wf8, bf2, W_cat, bc2]
    io_aliases = {}
    if s > 0:
        inputs.append(out_prev)
        in_specs.append(pl.BlockSpec(memory_space=pl.ANY))
        io_aliases = {6: 0}
    return pl.pallas_call(
        _mm_body,
        grid=(n_tiles,),
        in_specs=in_specs,
        out_specs=pl.BlockSpec((_TILE, _H), lambda i: (off + i, 0)),
        out_shape=jax.ShapeDtypeStruct((t, _H), jnp.float32),
        scratch_shapes=[
            pltpu.VMEM((8, _H), jnp.float32),
            pltpu.VMEM((1, _H), jnp.float32),
        ],
        input_output_aliases=io_aliases,
        compiler_params=pltpu.CompilerParams(
            dimension_semantics=("arbitrary",)),
    )(*inputs)


def kernel(input_sequence, emb_table, W_feat, b_feat, W_cat, b_cat):
    b, l, c = input_sequence.shape
    t = b * l
    seg = t // _SEGMENTS
    seq2 = input_sequence.reshape(t, c)
    ids = seq2[:, 0].astype(jnp.int32)
    seqt = seq2.T
    wf8 = jnp.concatenate(
        [jnp.zeros((1, _H), jnp.float32), W_feat,
         jnp.zeros((1, _H), jnp.float32)], axis=0)
    bf2 = b_feat.reshape(1, _H)
    bc2 = b_cat.reshape(1, _H)
    out = None
    for s in range(_SEGMENTS):
        g = _make_gather(t, seg, s)(emb_table, ids)
        out = _fused_matmul_seg(seqt, g, wf8, bf2, W_cat, bc2, out, s)
    return out.reshape(b, l, _H)


# speedup vs baseline: 1.0465x; 1.0465x over previous
import functools

import jax
import jax.numpy as jnp
from jax import lax
from jax.experimental import pallas as pl
from jax.experimental.pallas import tpu as pltpu
from jax.experimental.pallas import tpu_sc as plsc

_H = 128
_SEGMENTS = 2
_TILE = 6400


@functools.lru_cache(maxsize=None)
def _make_gather(num_tokens, seg, s):
    info = plsc.get_sparse_core_info()
    nc, ns = info.num_cores, info.num_subcores
    nw = nc * ns
    assert seg % nw == 0
    b_per_w = seg // nw
    chunk = 400
    assert b_per_w % chunk == 0
    n_chunks = b_per_w // chunk
    mesh = plsc.VectorSubcoreMesh(core_axis_name="c", subcore_axis_name="s")

    @functools.partial(
        pl.kernel,
        mesh=mesh,
        out_type=jax.ShapeDtypeStruct((seg, _H), jnp.float32),
        scratch_types=[
            pltpu.VMEM((chunk,), jnp.int32),
            pltpu.VMEM((chunk,), jnp.int32),
            pltpu.VMEM((chunk, _H), jnp.float32),
            pltpu.VMEM((chunk, _H), jnp.float32),
            pltpu.SemaphoreType.DMA,
            pltpu.SemaphoreType.DMA,
            pltpu.SemaphoreType.DMA,
            pltpu.SemaphoreType.DMA,
            pltpu.SemaphoreType.DMA,
            pltpu.SemaphoreType.DMA,
        ],
    )
    def gather(table_hbm, ids_hbm, out_hbm, idx0, idx1, rows0, rows1,
               isem0, isem1, gsem0, gsem1, wsem0, wsem1):
        wid = lax.axis_index("s") * nc + lax.axis_index("c")
        base = wid * b_per_w
        idxs = (idx0, idx1)
        bufs = (rows0, rows1)
        isems = (isem0, isem1)
        gsems = (gsem0, gsem1)
        wsems = (wsem0, wsem1)
        ih = [None] * n_chunks
        gh = [None] * n_chunks
        wh = [None] * n_chunks

        def start_i(i):
            ih[i] = pltpu.async_copy(
                ids_hbm.at[pl.ds(s * seg + base + i * chunk, chunk)],
                idxs[i % 2], isems[i % 2])

        def start_g(i):
            gh[i] = pltpu.async_copy(
                table_hbm.at[idxs[i % 2]], bufs[i % 2], gsems[i % 2])

        def start_w(i):
            wh[i] = pltpu.async_copy(
                bufs[i % 2], out_hbm.at[pl.ds(base + i * chunk, chunk)],
                wsems[i % 2])

        start_i(0)
        ih[0].wait()
        start_g(0)
        if n_chunks > 1:
            start_i(1)
        for i in range(n_chunks):
            gh[i].wait()
            if i >= 1:
                wh[i - 1].wait()
            if i + 2 < n_chunks:
                start_i(i + 2)
            if i + 1 < n_chunks:
                ih[i + 1].wait()
                start_g(i + 1)
            start_w(i)
        wh[n_chunks - 1].wait()

    return gather


def _mm_body(seqt_ref, g_ref, wf_ref, bf_ref, wc_ref, bc_ref, *rest):
    out_ref, weff_ref, beff_ref = rest[-3], rest[-2], rest[-1]
    wtop = wc_ref[:_H, :]

    @pl.when(pl.program_id(0) == 0)
    def _fold_weights():
        weff_ref[...] = lax.dot(
            wf_ref[...], wtop, precision=lax.Precision.HIGHEST,
            preferred_element_type=jnp.float32)
        beff_ref[...] = lax.dot(
            bf_ref[...], wtop, precision=lax.Precision.HIGHEST,
            preferred_element_type=jnp.float32) + bc_ref[...]

    wbot = wc_ref[_H:, :]
    out_ref[...] = (
        lax.dot_general(seqt_ref[...], weff_ref[:7, :],
                        (((0,), (0,)), ((), ())),
                        precision=lax.Precision.HIGHEST,
                        preferred_element_type=jnp.float32)
        + lax.dot(g_ref[...].astype(jnp.bfloat16),
                  wbot.astype(jnp.bfloat16),
                  preferred_element_type=jnp.float32)
        + beff_ref[...])


def _fused_matmul_seg(seqt, g, wf8, bf2, W_cat, bc2, out_prev, s):
    t = seqt.shape[1]
    seg = g.shape[0]
    assert seg % _TILE == 0
    n_tiles = seg // _TILE
    off = s * n_tiles
    in_specs = [
        pl.BlockSpec((7, _TILE), lambda i: (0, off + i)),
        pl.BlockSpec((_TILE, _H), lambda i: (i, 0)),
        pl.BlockSpec((8, _H), lambda i: (0, 0)),
        pl.BlockSpec((1, _H), lambda i: (0, 0)),
        pl.BlockSpec((2 * _H, _H), lambda i: (0, 0)),
        pl.BlockSpec((1, _H), lambda i: (0, 0)),
    ]
    inputs = [seqt, g, wf8, bf2, W_cat, bc2]
    io_aliases = {}
    if s > 0:
        inputs.append(out_prev)
        in_specs.append(pl.BlockSpec(memory_space=pl.ANY))
        io_aliases = {6: 0}
    return pl.pallas_call(
        _mm_body,
        grid=(n_tiles,),
        in_specs=in_specs,
        out_specs=pl.BlockSpec((_TILE, _H), lambda i: (off + i, 0)),
        out_shape=jax.ShapeDtypeStruct((t, _H), jnp.float32),
        scratch_shapes=[
            pltpu.VMEM((8, _H), jnp.float32),
            pltpu.VMEM((1, _H), jnp.float32),
        ],
        input_output_aliases=io_aliases,
        compiler_params=pltpu.CompilerParams(
            dimension_semantics=("arbitrary",)),
    )(*inputs)


def kernel(input_sequence, emb_table, W_feat, b_feat, W_cat, b_cat):
    b, l, c = input_sequence.shape
    t = b * l
    seg = t // _SEGMENTS
    seq2 = input_sequence.reshape(t, c)
    ids = seq2[:, 0].astype(jnp.int32)
    seqt = seq2.T
    wf8 = jnp.concatenate(
        [jnp.zeros((1, _H), jnp.float32), W_feat,
         jnp.zeros((1, _H), jnp.float32)], axis=0)
    bf2 = b_feat.reshape(1, _H)
    bc2 = b_cat.reshape(1, _H)
    out = None
    for s in range(_SEGMENTS):
        g = _make_gather(t, seg, s)(emb_table, ids)
        out = _fused_matmul_seg(seqt, g, wf8, bf2, W_cat, bc2, out, s)
    return out.reshape(b, l, _H)


# speedup vs baseline: 1.0626x; 1.0154x over previous
import functools

import jax
import jax.numpy as jnp
from jax import lax
from jax.experimental import pallas as pl
from jax.experimental.pallas import tpu as pltpu
from jax.experimental.pallas import tpu_sc as plsc

_H = 128
_SEGMENTS = 2
_TILE = 12800


@functools.lru_cache(maxsize=None)
def _make_gather(num_tokens, seg, s):
    info = plsc.get_sparse_core_info()
    nc, ns = info.num_cores, info.num_subcores
    nw = nc * ns
    assert seg % nw == 0
    b_per_w = seg // nw
    chunk = 400
    assert b_per_w % chunk == 0
    n_chunks = b_per_w // chunk
    mesh = plsc.VectorSubcoreMesh(core_axis_name="c", subcore_axis_name="s")

    @functools.partial(
        pl.kernel,
        mesh=mesh,
        out_type=jax.ShapeDtypeStruct((seg, _H), jnp.float32),
        scratch_types=[
            pltpu.VMEM((chunk,), jnp.int32),
            pltpu.VMEM((chunk,), jnp.int32),
            pltpu.VMEM((chunk, _H), jnp.float32),
            pltpu.VMEM((chunk, _H), jnp.float32),
            pltpu.SemaphoreType.DMA,
            pltpu.SemaphoreType.DMA,
            pltpu.SemaphoreType.DMA,
            pltpu.SemaphoreType.DMA,
            pltpu.SemaphoreType.DMA,
            pltpu.SemaphoreType.DMA,
        ],
    )
    def gather(table_hbm, ids_hbm, out_hbm, idx0, idx1, rows0, rows1,
               isem0, isem1, gsem0, gsem1, wsem0, wsem1):
        wid = lax.axis_index("s") * nc + lax.axis_index("c")
        base = wid * b_per_w
        idxs = (idx0, idx1)
        bufs = (rows0, rows1)
        isems = (isem0, isem1)
        gsems = (gsem0, gsem1)
        wsems = (wsem0, wsem1)
        ih = [None] * n_chunks
        gh = [None] * n_chunks
        wh = [None] * n_chunks

        def start_i(i):
            ih[i] = pltpu.async_copy(
                ids_hbm.at[pl.ds(s * seg + base + i * chunk, chunk)],
                idxs[i % 2], isems[i % 2])

        def start_g(i):
            gh[i] = pltpu.async_copy(
                table_hbm.at[idxs[i % 2]], bufs[i % 2], gsems[i % 2])

        def start_w(i):
            wh[i] = pltpu.async_copy(
                bufs[i % 2], out_hbm.at[pl.ds(base + i * chunk, chunk)],
                wsems[i % 2])

        start_i(0)
        ih[0].wait()
        start_g(0)
        if n_chunks > 1:
            start_i(1)
        for i in range(n_chunks):
            gh[i].wait()
            if i >= 1:
                wh[i - 1].wait()
            if i + 2 < n_chunks:
                start_i(i + 2)
            if i + 1 < n_chunks:
                ih[i + 1].wait()
                start_g(i + 1)
            start_w(i)
        wh[n_chunks - 1].wait()

    return gather


def _mm_body(seqt_ref, g_ref, wf_ref, bf_ref, wc_ref, bc_ref, *rest):
    out_ref, weff_ref, beff_ref = rest[-3], rest[-2], rest[-1]
    wtop = wc_ref[:_H, :]

    @pl.when(pl.program_id(0) == 0)
    def _fold_weights():
        weff_ref[...] = lax.dot(
            wf_ref[...], wtop, precision=lax.Precision.HIGHEST,
            preferred_element_type=jnp.float32)
        beff_ref[...] = lax.dot(
            bf_ref[...], wtop, precision=lax.Precision.HIGHEST,
            preferred_element_type=jnp.float32) + bc_ref[...]

    wbot = wc_ref[_H:, :]
    out_ref[...] = (
        lax.dot_general(seqt_ref[...], weff_ref[:7, :],
                        (((0,), (0,)), ((), ())),
                        precision=lax.Precision.HIGHEST,
                        preferred_element_type=jnp.float32)
        + lax.dot(g_ref[...].astype(jnp.bfloat16),
                  wbot.astype(jnp.bfloat16),
                  preferred_element_type=jnp.float32)
        + beff_ref[...])


def _fused_matmul_seg(seqt, g, wf8, bf2, W_cat, bc2, out_prev, s):
    t = seqt.shape[1]
    seg = g.shape[0]
    assert seg % _TILE == 0
    n_tiles = seg // _TILE
    off = s * n_tiles
    in_specs = [
        pl.BlockSpec((7, _TILE), lambda i: (0, off + i)),
        pl.BlockSpec((_TILE, _H), lambda i: (i, 0)),
        pl.BlockSpec((8, _H), lambda i: (0, 0)),
        pl.BlockSpec((1, _H), lambda i: (0, 0)),
        pl.BlockSpec((2 * _H, _H), lambda i: (0, 0)),
        pl.BlockSpec((1, _H), lambda i: (0, 0)),
    ]
    inputs = [seqt, g, wf8, bf2, W_cat, bc2]
    io_aliases = {}
    if s > 0:
        inputs.append(out_prev)
        in_specs.append(pl.BlockSpec(memory_space=pl.ANY))
        io_aliases = {6: 0}
    return pl.pallas_call(
        _mm_body,
        grid=(n_tiles,),
        in_specs=in_specs,
        out_specs=pl.BlockSpec((_TILE, _H), lambda i: (off + i, 0)),
        out_shape=jax.ShapeDtypeStruct((t, _H), jnp.float32),
        scratch_shapes=[
            pltpu.VMEM((8, _H), jnp.float32),
            pltpu.VMEM((1, _H), jnp.float32),
        ],
        input_output_aliases=io_aliases,
        compiler_params=pltpu.CompilerParams(
            dimension_semantics=("arbitrary",)),
    )(*inputs)


def kernel(input_sequence, emb_table, W_feat, b_feat, W_cat, b_cat):
    b, l, c = input_sequence.shape
    t = b * l
    seg = t // _SEGMENTS
    seq2 = input_sequence.reshape(t, c)
    ids = seq2[:, 0].astype(jnp.int32)
    seqt = seq2.T
    wf8 = jnp.concatenate(
        [jnp.zeros((1, _H), jnp.float32), W_feat,
         jnp.zeros((1, _H), jnp.float32)], axis=0)
    bf2 = b_feat.reshape(1, _H)
    bc2 = b_cat.reshape(1, _H)
    out = None
    for s in range(_SEGMENTS):
        g = _make_gather(t, seg, s)(emb_table, ids)
        out = _fused_matmul_seg(seqt, g, wf8, bf2, W_cat, bc2, out, s)
    return out.reshape(b, l, _H)


# speedup vs baseline: 1.1861x; 1.1162x over previous
import functools

import jax
import jax.numpy as jnp
from jax import lax
from jax.experimental import pallas as pl
from jax.experimental.pallas import tpu as pltpu
from jax.experimental.pallas import tpu_sc as plsc

_H = 128
_SEGMENTS = 2
_TILE = 12800


@functools.lru_cache(maxsize=None)
def _make_gather(num_tokens, seg, s):
    info = plsc.get_sparse_core_info()
    nc, ns = info.num_cores, info.num_subcores
    nw = nc * ns
    assert seg % nw == 0
    b_per_w = seg // nw
    chunk = 400
    assert b_per_w % chunk == 0
    n_chunks = b_per_w // chunk
    mesh = plsc.VectorSubcoreMesh(core_axis_name="c", subcore_axis_name="s")

    @functools.partial(
        pl.kernel,
        mesh=mesh,
        out_type=jax.ShapeDtypeStruct((seg, _H), jnp.float32),
        scratch_types=[
            pltpu.VMEM((chunk,), jnp.int32),
            pltpu.VMEM((chunk,), jnp.int32),
            pltpu.VMEM((chunk, _H), jnp.float32),
            pltpu.VMEM((chunk, _H), jnp.float32),
            pltpu.SemaphoreType.DMA,
            pltpu.SemaphoreType.DMA,
            pltpu.SemaphoreType.DMA,
            pltpu.SemaphoreType.DMA,
            pltpu.SemaphoreType.DMA,
            pltpu.SemaphoreType.DMA,
        ],
    )
    def gather(table_hbm, ids_hbm, out_hbm, idx0, idx1, rows0, rows1,
               isem0, isem1, gsem0, gsem1, wsem0, wsem1):
        wid = lax.axis_index("s") * nc + lax.axis_index("c")
        base = wid * b_per_w
        idxs = (idx0, idx1)
        bufs = (rows0, rows1)
        isems = (isem0, isem1)
        gsems = (gsem0, gsem1)
        wsems = (wsem0, wsem1)
        ih = [None] * n_chunks
        gh = [None] * n_chunks
        wh = [None] * n_chunks

        def start_i(i):
            ih[i] = pltpu.async_copy(
                ids_hbm.at[pl.ds(s * seg + base + i * chunk, chunk)],
                idxs[i % 2], isems[i % 2])

        def start_g(i):
            gh[i] = pltpu.async_copy(
                table_hbm.at[idxs[i % 2]], bufs[i % 2], gsems[i % 2])

        def start_w(i):
            wh[i] = pltpu.async_copy(
                bufs[i % 2], out_hbm.at[pl.ds(base + i * chunk, chunk)],
                wsems[i % 2])

        start_i(0)
        ih[0].wait()
        start_g(0)
        if n_chunks > 1:
            start_i(1)
        for i in range(n_chunks):
            gh[i].wait()
            if i >= 1:
                wh[i - 1].wait()
            if i + 2 < n_chunks:
                start_i(i + 2)
            if i + 1 < n_chunks:
                ih[i + 1].wait()
                start_g(i + 1)
            start_w(i)
        wh[n_chunks - 1].wait()

    return gather


def _mm_body(seqt_ref, g_ref, wf_ref, bf_ref, wc_ref, bc_ref, *rest):
    out_ref, weff_ref, beff_ref = rest[-3], rest[-2], rest[-1]
    wtop = wc_ref[:_H, :]

    @pl.when(pl.program_id(0) == 0)
    def _fold_weights():
        weff_ref[...] = lax.dot(
            wf_ref[...], wtop, precision=lax.Precision.HIGHEST,
            preferred_element_type=jnp.float32)
        beff_ref[...] = lax.dot(
            bf_ref[...], wtop, precision=lax.Precision.HIGHEST,
            preferred_element_type=jnp.float32) + bc_ref[...]

    wbot = wc_ref[_H:, :]
    out_ref[...] = (
        lax.dot_general(seqt_ref[...], weff_ref[:7, :],
                        (((0,), (0,)), ((), ())),
                        precision=lax.Precision.DEFAULT,
                        preferred_element_type=jnp.float32)
        + lax.dot(g_ref[...].astype(jnp.bfloat16),
                  wbot.astype(jnp.bfloat16),
                  preferred_element_type=jnp.float32)
        + beff_ref[...])


def _fused_matmul_seg(seqt, g, wf8, bf2, W_cat, bc2, out_prev, s):
    t = seqt.shape[1]
    seg = g.shape[0]
    assert seg % _TILE == 0
    n_tiles = seg // _TILE
    off = s * n_tiles
    in_specs = [
        pl.BlockSpec((7, _TILE), lambda i: (0, off + i)),
        pl.BlockSpec((_TILE, _H), lambda i: (i, 0)),
        pl.BlockSpec((8, _H), lambda i: (0, 0)),
        pl.BlockSpec((1, _H), lambda i: (0, 0)),
        pl.BlockSpec((2 * _H, _H), lambda i: (0, 0)),
        pl.BlockSpec((1, _H), lambda i: (0, 0)),
    ]
    inputs = [seqt, g, wf8, bf2, W_cat, bc2]
    io_aliases = {}
    if s > 0:
        inputs.append(out_prev)
        in_specs.append(pl.BlockSpec(memory_space=pl.ANY))
        io_aliases = {6: 0}
    return pl.pallas_call(
        _mm_body,
        grid=(n_tiles,),
        in_specs=in_specs,
        out_specs=pl.BlockSpec((_TILE, _H), lambda i: (off + i, 0)),
        out_shape=jax.ShapeDtypeStruct((t, _H), jnp.float32),
        scratch_shapes=[
            pltpu.VMEM((8, _H), jnp.float32),
            pltpu.VMEM((1, _H), jnp.float32),
        ],
        input_output_aliases=io_aliases,
        compiler_params=pltpu.CompilerParams(
            dimension_semantics=("arbitrary",)),
    )(*inputs)


def kernel(input_sequence, emb_table, W_feat, b_feat, W_cat, b_cat):
    b, l, c = input_sequence.shape
    t = b * l
    seg = t // _SEGMENTS
    seq2 = input_sequence.reshape(t, c)
    ids = seq2[:, 0].astype(jnp.int32)
    seqt = seq2.T
    wf8 = jnp.concatenate(
        [jnp.zeros((1, _H), jnp.float32), W_feat,
         jnp.zeros((1, _H), jnp.float32)], axis=0)
    bf2 = b_feat.reshape(1, _H)
    bc2 = b_cat.reshape(1, _H)
    out = None
    for s in range(_SEGMENTS):
        g = _make_gather(t, seg, s)(emb_table, ids)
        out = _fused_matmul_seg(seqt, g, wf8, bf2, W_cat, bc2, out, s)
    return out.reshape(b, l, _H)


# speedup vs baseline: 1.1883x; 1.0019x over previous
import functools

import jax
import jax.numpy as jnp
from jax import lax
from jax.experimental import pallas as pl
from jax.experimental.pallas import tpu as pltpu
from jax.experimental.pallas import tpu_sc as plsc

_H = 128
_SEGMENTS = 2
_TILE = 12800


@functools.lru_cache(maxsize=None)
def _make_gather(num_tokens, seg, s):
    info = plsc.get_sparse_core_info()
    nc, ns = info.num_cores, info.num_subcores
    nw = nc * ns
    assert seg % nw == 0
    b_per_w = seg // nw
    chunk = 400
    assert b_per_w % chunk == 0
    n_chunks = b_per_w // chunk
    mesh = plsc.VectorSubcoreMesh(core_axis_name="c", subcore_axis_name="s")

    @functools.partial(
        pl.kernel,
        mesh=mesh,
        out_type=jax.ShapeDtypeStruct((seg, _H), jnp.float32),
        scratch_types=[
            pltpu.VMEM((chunk,), jnp.int32),
            pltpu.VMEM((chunk,), jnp.int32),
            pltpu.VMEM((chunk, _H), jnp.float32),
            pltpu.VMEM((chunk, _H), jnp.float32),
            pltpu.SemaphoreType.DMA,
            pltpu.SemaphoreType.DMA,
            pltpu.SemaphoreType.DMA,
            pltpu.SemaphoreType.DMA,
            pltpu.SemaphoreType.DMA,
            pltpu.SemaphoreType.DMA,
        ],
    )
    def gather(table_hbm, ids_hbm, out_hbm, idx0, idx1, rows0, rows1,
               isem0, isem1, gsem0, gsem1, wsem0, wsem1):
        wid = lax.axis_index("s") * nc + lax.axis_index("c")
        base = wid * b_per_w
        idxs = (idx0, idx1)
        bufs = (rows0, rows1)
        isems = (isem0, isem1)
        gsems = (gsem0, gsem1)
        wsems = (wsem0, wsem1)
        ih = [None] * n_chunks
        gh = [None] * n_chunks
        wh = [None] * n_chunks

        def start_i(i):
            ih[i] = pltpu.async_copy(
                ids_hbm.at[pl.ds(s * seg + base + i * chunk, chunk)],
                idxs[i % 2], isems[i % 2])

        def start_g(i):
            gh[i] = pltpu.async_copy(
                table_hbm.at[idxs[i % 2]], bufs[i % 2], gsems[i % 2])

        def start_w(i):
            wh[i] = pltpu.async_copy(
                bufs[i % 2], out_hbm.at[pl.ds(base + i * chunk, chunk)],
                wsems[i % 2])

        start_i(0)
        ih[0].wait()
        start_g(0)
        if n_chunks > 1:
            start_i(1)
        for i in range(n_chunks):
            gh[i].wait()
            if i >= 1:
                wh[i - 1].wait()
            if i + 2 < n_chunks:
                start_i(i + 2)
            if i + 1 < n_chunks:
                ih[i + 1].wait()
                start_g(i + 1)
            start_w(i)
        wh[n_chunks - 1].wait()

    return gather


def _mm_body(seqt_ref, g_ref, wf_ref, bf_ref, wc_ref, bc_ref, *rest):
    out_ref, weff_ref, beff_ref = rest[-3], rest[-2], rest[-1]
    wtop = wc_ref[:_H, :]

    @pl.when(pl.program_id(0) == 0)
    def _fold_weights():
        weff_ref[...] = lax.dot(
            wf_ref[...], wtop, precision=lax.Precision.HIGHEST,
            preferred_element_type=jnp.float32)
        beff_ref[...] = lax.dot(
            bf_ref[...], wtop, precision=lax.Precision.HIGHEST,
            preferred_element_type=jnp.float32) + bc_ref[...]

    wbot = wc_ref[_H:, :]
    out_ref[...] = (
        lax.dot_general(seqt_ref[...], weff_ref[:7, :],
                        (((0,), (0,)), ((), ())),
                        precision=lax.Precision.DEFAULT,
                        preferred_element_type=jnp.float32)
        + lax.dot(g_ref[...], wbot,
                  preferred_element_type=jnp.float32)
        + beff_ref[...])


def _fused_matmul_seg(seqt, g, wf8, bf2, W_cat, bc2, out_prev, s):
    t = seqt.shape[1]
    seg = g.shape[0]
    assert seg % _TILE == 0
    n_tiles = seg // _TILE
    off = s * n_tiles
    in_specs = [
        pl.BlockSpec((7, _TILE), lambda i: (0, off + i)),
        pl.BlockSpec((_TILE, _H), lambda i: (i, 0)),
        pl.BlockSpec((8, _H), lambda i: (0, 0)),
        pl.BlockSpec((1, _H), lambda i: (0, 0)),
        pl.BlockSpec((2 * _H, _H), lambda i: (0, 0)),
        pl.BlockSpec((1, _H), lambda i: (0, 0)),
    ]
    inputs = [seqt, g, wf8, bf2, W_cat, bc2]
    io_aliases = {}
    if s > 0:
        inputs.append(out_prev)
        in_specs.append(pl.BlockSpec(memory_space=pl.ANY))
        io_aliases = {6: 0}
    return pl.pallas_call(
        _mm_body,
        grid=(n_tiles,),
        in_specs=in_specs,
        out_specs=pl.BlockSpec((_TILE, _H), lambda i: (off + i, 0)),
        out_shape=jax.ShapeDtypeStruct((t, _H), jnp.float32),
        scratch_shapes=[
            pltpu.VMEM((8, _H), jnp.float32),
            pltpu.VMEM((1, _H), jnp.float32),
        ],
        input_output_aliases=io_aliases,
        compiler_params=pltpu.CompilerParams(
            dimension_semantics=("arbitrary",)),
    )(*inputs)


def kernel(input_sequence, emb_table, W_feat, b_feat, W_cat, b_cat):
    b, l, c = input_sequence.shape
    t = b * l
    seg = t // _SEGMENTS
    seq2 = input_sequence.reshape(t, c)
    ids = seq2[:, 0].astype(jnp.int32)
    seqt = seq2.T
    wf8 = jnp.concatenate(
        [jnp.zeros((1, _H), jnp.float32), W_feat,
         jnp.zeros((1, _H), jnp.float32)], axis=0)
    bf2 = b_feat.reshape(1, _H)
    bc2 = b_cat.reshape(1, _H)
    out = None
    for s in range(_SEGMENTS):
        g = _make_gather(t, seg, s)(emb_table, ids)
        out = _fused_matmul_seg(seqt, g, wf8, bf2, W_cat, bc2, out, s)
    return out.reshape(b, l, _H)
